# baseline (speedup 1.0000x reference)
import functools

import jax
import jax.numpy as jnp
from jax import lax
from jax.experimental import pallas as pl
from jax.experimental.pallas import tpu as pltpu
from jax.experimental.pallas import tpu_sc as plsc

_NC, _NS, _L = 2, 16, 16
_NW = _NC * _NS


def _rsqrt_nr(d):
    i = plsc.bitcast(d, jnp.int32)
    i = jnp.int32(0x5F3759DF) - lax.shift_right_logical(i, 1)
    y = plsc.bitcast(i, jnp.float32)
    for _ in range(3):
        y = y * (1.5 - 0.5 * d * y * y)
    return jnp.where(d > 0.0, y, 0.0)


def _make_deg(E, NPAD):
    EPT = E // _NW
    NPS = NPAD // _NS
    mesh = plsc.VectorSubcoreMesh(
        core_axis_name="c", subcore_axis_name="s",
        num_cores=_NC, num_subcores=_NS)

    @functools.partial(
        pl.kernel,
        out_type=jax.ShapeDtypeStruct((_NC, NPAD), jnp.float32),
        mesh=mesh,
        scratch_types=[
            pltpu.VMEM((EPT,), jnp.int32),
            pltpu.VMEM((EPT,), jnp.float32),
            pltpu.VMEM((NPAD,), jnp.float32),
            pltpu.VMEM_SHARED((_NS, NPAD), jnp.float32),
            pltpu.VMEM((_NS, NPS), jnp.float32),
            pltpu.VMEM((NPS,), jnp.float32),
        ],
        compiler_params=pltpu.CompilerParams(needs_layout_passes=False),
    )
    def deg_kernel(col_hbm, ew_hbm, deg2_hbm, col_v, ew_v, part_v, stage_sh,
                   gath_v, out_v):
        c = lax.axis_index("c")
        s = lax.axis_index("s")
        gid = c * _NS + s
        zv = jnp.zeros((_L,), jnp.float32)

        def zbody(i, _):
            part_v[pl.ds(i * _L, _L)] = zv
            return 0
        lax.fori_loop(0, NPAD // _L, zbody, 0, unroll=4)

        pltpu.sync_copy(col_hbm.at[pl.ds(gid * EPT, EPT)], col_v)
        pltpu.sync_copy(ew_hbm.at[pl.ds(gid * EPT, EPT)], ew_v)

        def ebody(i, _):
            cv = col_v[pl.ds(i * _L, _L)]
            wv = ew_v[pl.ds(i * _L, _L)]
            plsc.addupdate_scatter(part_v, [cv], wv)
            return 0
        lax.fori_loop(0, EPT // _L, ebody, 0, unroll=4)

        pltpu.sync_copy(part_v, stage_sh.at[s])
        plsc.subcore_barrier()
        for t in range(_NS):
            pltpu.sync_copy(stage_sh.at[t, pl.ds(s * NPS, NPS)], gath_v.at[t])

        def cbody(j, _):
            acc = gath_v[0, pl.ds(j * _L, _L)]
            for t in range(1, _NS):
                acc = acc + gath_v[t, pl.ds(j * _L, _L)]
            out_v[pl.ds(j * _L, _L)] = acc
            return 0
        lax.fori_loop(0, NPS // _L, cbody, 0)
        pltpu.sync_copy(out_v, deg2_hbm.at[c, pl.ds(s * NPS, NPS)])

    return deg_kernel


def _make_agg(E, N, D, NPAD, C):
    EPT = E // _NW
    EPC = EPT // C
    NCHB = 24
    B = C * NCHB
    NBLK = EPT // B
    assert EPT % B == 0
    NTRI = NCHB // 3
    RPT = NPAD // _NS
    assert RPT % C == 0
    NWS = RPT // C
    DCH = 640 if NPAD % 640 == 0 else NPAD
    SB = max(B, 2 * DCH)
    NPS = NPAD // _NS
    mesh = plsc.VectorSubcoreMesh(
        core_axis_name="c", subcore_axis_name="s",
        num_cores=_NC, num_subcores=_NS)

    @functools.partial(
        pl.kernel,
        out_type=(jax.ShapeDtypeStruct((_NC, NPAD, D), jnp.float32),
                  jax.ShapeDtypeStruct((NPAD,), jnp.float32)),
        mesh=mesh,
        scratch_types=[
            pltpu.VMEM((NPAD,), jnp.float32),
            pltpu.VMEM((NCHB, C), jnp.int32),
            pltpu.VMEM((B,), jnp.int32),
            pltpu.VMEM((SB,), jnp.float32),
            pltpu.VMEM((C,), jnp.int32),
            pltpu.VMEM((C, D), jnp.float32),
            pltpu.VMEM((C, D), jnp.float32),
            pltpu.VMEM((C, D), jnp.float32),
            pltpu.VMEM_SHARED((NPAD, D), jnp.float32),
            pltpu.SemaphoreType.DMA,
            pltpu.SemaphoreType.DMA,
            pltpu.SemaphoreType.DMA,
            pltpu.SemaphoreType.DMA,
            pltpu.SemaphoreType.DMA,
            pltpu.SemaphoreType.DMA,
        ],
        compiler_params=pltpu.CompilerParams(needs_layout_passes=False),
    )
    def agg_kernel(row_hbm, colr_hbm, ew_hbm, deg2_hbm, out0_hbm,
                   part_hbm, dis_hbm,
                   dis_v, colb_v, rowb_v, sb_v, zidx_v, msg0_v, msg1_v, msg2_v,
                   agg_sh, gsem0, gsem1, gsem2, ssem0, ssem1, ssem2):
        c = lax.axis_index("c")
        s = lax.axis_index("s")
        gid = c * _NS + s
        bufs = ((msg0_v, gsem0, ssem0),
                (msg1_v, gsem1, ssem1),
                (msg2_v, gsem2, ssem2))

        def dchunk(b, _):
            pltpu.sync_copy(deg2_hbm.at[0, pl.ds(b * DCH, DCH)],
                            sb_v.at[pl.ds(0, DCH)])
            pltpu.sync_copy(deg2_hbm.at[1, pl.ds(b * DCH, DCH)],
                            sb_v.at[pl.ds(DCH, DCH)])

            def dbody(j, _):
                d = sb_v[pl.ds(j * _L, _L)] + sb_v[pl.ds(DCH + j * _L, _L)]
                dis_v[pl.ds(b * DCH + j * _L, _L)] = _rsqrt_nr(d)
                return 0
            lax.fori_loop(0, DCH // _L, dbody, 0)
            return 0
        lax.fori_loop(0, NPAD // DCH, dchunk, 0)

        @pl.when(c == 0)
        def _():
            pltpu.sync_copy(dis_v.at[pl.ds(s * NPS, NPS)],
                            dis_hbm.at[pl.ds(s * NPS, NPS)])

        zv = jnp.zeros((_L,), jnp.float32)
        zvi = jnp.zeros((_L,), jnp.int32)

        def zrow(i, _):
            for kk in range(D // _L):
                msg0_v[i, pl.ds(kk * _L, _L)] = zv
                msg1_v[i, pl.ds(kk * _L, _L)] = zv
                msg2_v[i, pl.ds(kk * _L, _L)] = zv
            return 0
        lax.fori_loop(0, C, zrow, 0)
        for i in range(C // _L):
            zidx_v[pl.ds(i * _L, _L)] = zvi

        def zagg(i, _):
            pltpu.sync_copy(msg0_v, agg_sh.at[pl.ds(s * RPT + i * C, C)])
            return 0
        lax.fori_loop(0, NWS, zagg, 0)

        plsc.subcore_barrier()

        base = gid * EPT

        def _mult(msg, k):
            def gbody(g, _):
                sv = sb_v[pl.ds(k * C + g * _L, _L)]
                for j in range(_L):
                    sc = sv[j]
                    for kk in range(D // _L):
                        msg[g * _L + j, pl.ds(kk * _L, _L)] = (
                            msg[g * _L + j, pl.ds(kk * _L, _L)] * sc)
                return 0
            lax.fori_loop(0, C // _L, gbody, 0)

        def _gissue(k, mb, gs):
            pltpu.async_copy(out0_hbm.at[rowb_v.at[pl.ds(k * C, C)]], mb, gs)

        def _gwait(k, mb, gs):
            pltpu.make_async_copy(
                out0_hbm.at[rowb_v.at[pl.ds(k * C, C)]], mb, gs).wait()

        def _swait(ss):
            pltpu.make_async_copy(msg0_v, agg_sh.at[zidx_v], ss).wait()

        def _stage(k, q):
            mb, gs, ss = bufs[q]
            _gwait(k, mb, gs)
            _mult(mb, k)
            pltpu.sync_copy(mb, agg_sh.at[colb_v.at[k]], add=True)
            pb, pgs, pss = bufs[(q + 2) % 3]
            _gissue(k + 2, pb, pgs)

        def block(bi, _):
            boff = base + bi * B
            pltpu.sync_copy(row_hbm.at[pl.ds(boff, B)], rowb_v)
            pltpu.sync_copy(ew_hbm.at[pl.ds(boff, B)], sb_v.at[pl.ds(0, B)])

            def sbody(i, _):
                rv = rowb_v[pl.ds(i * _L, _L)]
                dv = plsc.load_gather(dis_v, [rv])
                sb_v[pl.ds(i * _L, _L)] = dv * sb_v[pl.ds(i * _L, _L)]
                return 0
            lax.fori_loop(0, B // _L, sbody, 0)

            pltpu.sync_copy(colr_hbm.at[gid, pl.ds(bi * NCHB, NCHB)], colb_v)

            _gissue(0, msg0_v, gsem0)
            _gissue(1, msg1_v, gsem1)

            _stage(0, 0)
            _stage(1, 1)
            _stage(2, 2)

            def triple(t, _):
                k = 3 * t
                _stage(k, 0)
                _stage(k + 1, 1)
                _stage(k + 2, 2)
                return 0
            lax.fori_loop(1, NTRI - 1, triple, 0)

            kt = NCHB - 3
            _stage(kt, 0)
            for q, kk2 in ((1, kt + 1), (2, kt + 2)):
                mb, gs, ss = bufs[q]
                _gwait(kk2, mb, gs)
                _mult(mb, kk2)
                pltpu.sync_copy(mb, agg_sh.at[colb_v.at[kk2]], add=True)
            return 0
        lax.fori_loop(0, NBLK, block, 0)
        plsc.subcore_barrier()

        pltpu.async_copy(agg_sh.at[pl.ds(s * RPT, C)], msg0_v, gsem0)
        for k in range(NWS):
            cur, csem = (msg0_v, gsem0) if k % 2 == 0 else (msg1_v, gsem1)
            pltpu.make_async_copy(
                agg_sh.at[pl.ds(s * RPT + k * C, C)], cur, csem).wait()
            if k + 1 < NWS:
                nxt, nsem = (msg1_v, gsem1) if k % 2 == 0 else (msg0_v, gsem0)
                pltpu.async_copy(
                    agg_sh.at[pl.ds(s * RPT + (k + 1) * C, C)], nxt, nsem)
            pltpu.sync_copy(cur, part_hbm.at[c, pl.ds(s * RPT + k * C, C)])

    return agg_kernel


def _make_mm(N, D, BR):
    grid = N // BR

    def mm_body(x_ref, wpre_ref, bpre_ref, wlin_ref, blin_ref, wini_ref,
                wroot_ref, bias_ref, h1_ref, out0_ref, rootb_ref):
        xb = x_ref[...]
        h = lax.dot_general(xb, wpre_ref[...], (((1,), (1,)), ((), ())),
                            preferred_element_type=jnp.float32) + bpre_ref[...]
        h1 = lax.dot_general(h, wlin_ref[...], (((1,), (1,)), ((), ())),
                             preferred_element_type=jnp.float32) + blin_ref[...]
        h1 = jnp.where(h1 >= 0, h1, 0.01 * h1)
        h1_ref[...] = h1
        out0_ref[...] = jnp.dot(h1, wini_ref[...],
                                preferred_element_type=jnp.float32)
        rootb_ref[...] = jnp.dot(h1, wroot_ref[...],
                                 preferred_element_type=jnp.float32) + bias_ref[...]

    full = pl.BlockSpec((D, D), lambda i: (0, 0))
    bias = pl.BlockSpec((1, D), lambda i: (0, 0))
    rows = pl.BlockSpec((BR, D), lambda i: (i, 0))
    return pl.pallas_call(
        mm_body,
        grid=(grid,),
        in_specs=[rows, full, bias, full, bias, full, full, bias],
        out_specs=[rows, rows, rows],
        out_shape=[jax.ShapeDtypeStruct((N, D), jnp.float32)] * 3,
    )


def _make_epi(N, D, NPAD, BR):
    grid = N // BR
    rows = pl.BlockSpec((BR, D), lambda i: (i, 0))

    def epi_body(h1_ref, rootb_ref, p_ref, dis_ref, o_ref):
        ps = p_ref[0] + p_ref[1]
        a = jnp.maximum(ps * dis_ref[...] + rootb_ref[...], 0.0)
        o_ref[...] = jnp.tanh(h1_ref[...] + a)

    return pl.pallas_call(
        epi_body,
        grid=(grid,),
        in_specs=[
            rows, rows,
            pl.BlockSpec((2, BR, D), lambda i: (0, i, 0)),
            pl.BlockSpec((BR, 1), lambda i: (i, 0)),
        ],
        out_specs=rows,
        out_shape=jax.ShapeDtypeStruct((N, D), jnp.float32),
    )


def kernel(x, edge_index, edge_weight, W_pre, b_pre, W_lin, b_lin,
           arma_init_w, arma_root_w, arma_bias):
    N, D = x.shape
    E = edge_weight.shape[0]
    NPAD = ((N + _NW * _L - 1) // (_NW * _L)) * (_NW * _L)
    C = 64
    BLK = C * 24

    row = edge_index[0]
    col = edge_index[1]

    EPT = -(-E // (_NW * BLK)) * BLK
    EP = _NW * EPT
    if EP != E:
        pad = EP - E
        row_p = jnp.concatenate([row, jnp.zeros((pad,), jnp.int32)])
        col_p = jnp.concatenate([col, jnp.zeros((pad,), jnp.int32)])
        ew_p = jnp.concatenate([edge_weight, jnp.zeros((pad,), jnp.float32)])
    else:
        row_p, col_p, ew_p = row, col, edge_weight
    colr = col_p.reshape(_NW, EPT // C, C)

    BR = 1000 if N % 1000 == 0 else N
    deg2 = _make_deg(E, NPAD)(col, edge_weight)
    h1, out0, rootb = _make_mm(N, D, BR)(
        x, W_pre, b_pre.reshape(1, D), W_lin, b_lin.reshape(1, D),
        arma_init_w, arma_root_w, arma_bias.reshape(1, D))
    part, dis = _make_agg(EP, N, D, NPAD, C)(row_p, colr, ew_p, deg2, out0)
    dis2d = dis.reshape(NPAD, 1)
    return _make_epi(N, D, NPAD, BR)(h1, rootb, part, dis2d)


# speedup vs baseline: 4.6608x; 4.6608x over previous
import functools

import jax
import jax.numpy as jnp
from jax import lax
from jax.experimental import pallas as pl
from jax.experimental.pallas import tpu as pltpu
from jax.experimental.pallas import tpu_sc as plsc

_NC, _NS, _L = 2, 16, 16
_NW = _NC * _NS


def _rsqrt_nr(d):
    i = plsc.bitcast(d, jnp.int32)
    i = jnp.int32(0x5F3759DF) - lax.shift_right_logical(i, 1)
    y = plsc.bitcast(i, jnp.float32)
    for _ in range(3):
        y = y * (1.5 - 0.5 * d * y * y)
    return jnp.where(d > 0.0, y, 0.0)


def _make_deg(E, NPAD):
    EPT = E // _NW
    NPS = NPAD // _NS
    mesh = plsc.VectorSubcoreMesh(
        core_axis_name="c", subcore_axis_name="s",
        num_cores=_NC, num_subcores=_NS)

    @functools.partial(
        pl.kernel,
        out_type=jax.ShapeDtypeStruct((_NC, NPAD), jnp.float32),
        mesh=mesh,
        scratch_types=[
            pltpu.VMEM((EPT,), jnp.int32),
            pltpu.VMEM((EPT,), jnp.float32),
            pltpu.VMEM((NPAD,), jnp.float32),
            pltpu.VMEM_SHARED((_NS, NPAD), jnp.float32),
            pltpu.VMEM((_NS, NPS), jnp.float32),
            pltpu.VMEM((NPS,), jnp.float32),
        ],
        compiler_params=pltpu.CompilerParams(needs_layout_passes=False),
    )
    def deg_kernel(col_hbm, ew_hbm, deg2_hbm, col_v, ew_v, part_v, stage_sh,
                   gath_v, out_v):
        c = lax.axis_index("c")
        s = lax.axis_index("s")
        gid = c * _NS + s
        zv = jnp.zeros((_L,), jnp.float32)

        def zbody(i, _):
            part_v[pl.ds(i * _L, _L)] = zv
            return 0
        lax.fori_loop(0, NPAD // _L, zbody, 0, unroll=4)

        pltpu.sync_copy(col_hbm.at[pl.ds(gid * EPT, EPT)], col_v)
        pltpu.sync_copy(ew_hbm.at[pl.ds(gid * EPT, EPT)], ew_v)

        def ebody(i, _):
            cv = col_v[pl.ds(i * _L, _L)]
            wv = ew_v[pl.ds(i * _L, _L)]
            plsc.addupdate_scatter(part_v, [cv], wv)
            return 0
        lax.fori_loop(0, EPT // _L, ebody, 0, unroll=4)

        pltpu.sync_copy(part_v, stage_sh.at[s])
        plsc.subcore_barrier()
        for t in range(_NS):
            pltpu.sync_copy(stage_sh.at[t, pl.ds(s * NPS, NPS)], gath_v.at[t])

        def cbody(j, _):
            acc = gath_v[0, pl.ds(j * _L, _L)]
            for t in range(1, _NS):
                acc = acc + gath_v[t, pl.ds(j * _L, _L)]
            out_v[pl.ds(j * _L, _L)] = acc
            return 0
        lax.fori_loop(0, NPS // _L, cbody, 0)
        pltpu.sync_copy(out_v, deg2_hbm.at[c, pl.ds(s * NPS, NPS)])

    return deg_kernel


def _make_agg(E, N, D, NPAD, C):
    EPT = E // _NW
    EPC = EPT // C
    NCHB = 24
    B = C * NCHB
    NBLK = EPT // B
    assert EPT % B == 0
    NTRI = NCHB // 3
    RPT = NPAD // _NS
    assert RPT % C == 0
    NWS = RPT // C
    DCH = 640 if NPAD % 640 == 0 else NPAD
    SB = max(B, 2 * DCH)
    NPS = NPAD // _NS
    mesh = plsc.VectorSubcoreMesh(
        core_axis_name="c", subcore_axis_name="s",
        num_cores=_NC, num_subcores=_NS)

    @functools.partial(
        pl.kernel,
        out_type=(jax.ShapeDtypeStruct((_NC, NPAD, D), jnp.float32),
                  jax.ShapeDtypeStruct((NPAD,), jnp.float32)),
        mesh=mesh,
        scratch_types=[
            pltpu.VMEM((NPAD,), jnp.float32),
            pltpu.VMEM((NCHB, C), jnp.int32),
            pltpu.VMEM((B,), jnp.int32),
            pltpu.VMEM((SB,), jnp.float32),
            pltpu.VMEM((C,), jnp.int32),
            pltpu.VMEM((C, D), jnp.float32),
            pltpu.VMEM((C, D), jnp.float32),
            pltpu.VMEM((C, D), jnp.float32),
            pltpu.VMEM_SHARED((NPAD, D), jnp.float32),
            pltpu.SemaphoreType.DMA,
            pltpu.SemaphoreType.DMA,
            pltpu.SemaphoreType.DMA,
            pltpu.SemaphoreType.DMA,
            pltpu.SemaphoreType.DMA,
            pltpu.SemaphoreType.DMA,
        ],
        compiler_params=pltpu.CompilerParams(needs_layout_passes=False),
    )
    def agg_kernel(row_hbm, colr_hbm, ew_hbm, deg2_hbm, out0_hbm,
                   part_hbm, dis_hbm,
                   dis_v, colb_v, rowb_v, sb_v, zidx_v, msg0_v, msg1_v, msg2_v,
                   agg_sh, gsem0, gsem1, gsem2, ssem0, ssem1, ssem2):
        c = lax.axis_index("c")
        s = lax.axis_index("s")
        gid = c * _NS + s
        bufs = ((msg0_v, gsem0, ssem0),
                (msg1_v, gsem1, ssem1),
                (msg2_v, gsem2, ssem2))

        def dchunk(b, _):
            pltpu.sync_copy(deg2_hbm.at[0, pl.ds(b * DCH, DCH)],
                            sb_v.at[pl.ds(0, DCH)])
            pltpu.sync_copy(deg2_hbm.at[1, pl.ds(b * DCH, DCH)],
                            sb_v.at[pl.ds(DCH, DCH)])

            def dbody(j, _):
                d = sb_v[pl.ds(j * _L, _L)] + sb_v[pl.ds(DCH + j * _L, _L)]
                dis_v[pl.ds(b * DCH + j * _L, _L)] = _rsqrt_nr(d)
                return 0
            lax.fori_loop(0, DCH // _L, dbody, 0)
            return 0
        lax.fori_loop(0, NPAD // DCH, dchunk, 0)

        @pl.when(c == 0)
        def _():
            pltpu.sync_copy(dis_v.at[pl.ds(s * NPS, NPS)],
                            dis_hbm.at[pl.ds(s * NPS, NPS)])

        zv = jnp.zeros((_L,), jnp.float32)
        zvi = jnp.zeros((_L,), jnp.int32)

        def zrow(i, _):
            for kk in range(D // _L):
                msg0_v[i, pl.ds(kk * _L, _L)] = zv
                msg1_v[i, pl.ds(kk * _L, _L)] = zv
                msg2_v[i, pl.ds(kk * _L, _L)] = zv
            return 0
        lax.fori_loop(0, C, zrow, 0)
        for i in range(C // _L):
            zidx_v[pl.ds(i * _L, _L)] = zvi

        def zagg(i, _):
            pltpu.sync_copy(msg0_v, agg_sh.at[pl.ds(s * RPT + i * C, C)])
            return 0
        lax.fori_loop(0, NWS, zagg, 0)

        plsc.subcore_barrier()

        base = gid * EPT

        def _mult(msg, k):
            def gbody(g, _):
                sv = sb_v[pl.ds(k * C + g * _L, _L)]
                for j in range(_L):
                    sc = sv[j]
                    for kk in range(D // _L):
                        msg[g * _L + j, pl.ds(kk * _L, _L)] = (
                            msg[g * _L + j, pl.ds(kk * _L, _L)] * sc)
                return 0
            lax.fori_loop(0, C // _L, gbody, 0)

        def _gissue(k, mb, gs):
            pltpu.async_copy(out0_hbm.at[rowb_v.at[pl.ds(k * C, C)]], mb, gs)

        def _gwait(k, mb, gs):
            pltpu.make_async_copy(
                out0_hbm.at[rowb_v.at[pl.ds(k * C, C)]], mb, gs).wait()

        def _swait(ss):
            pltpu.make_async_copy(msg0_v, agg_sh.at[zidx_v], ss).wait()

        def _stage(k, q):
            mb, gs, ss = bufs[q]
            _gwait(k, mb, gs)
            _mult(mb, k)
            pltpu.sync_copy(mb, agg_sh.at[colb_v.at[k]], add=True)
            pb, pgs, pss = bufs[(q + 2) % 3]
            _gissue(k + 2, pb, pgs)

        def block(bi, _):
            boff = base + bi * B
            pltpu.sync_copy(row_hbm.at[pl.ds(boff, B)], rowb_v)
            pltpu.sync_copy(ew_hbm.at[pl.ds(boff, B)], sb_v.at[pl.ds(0, B)])

            def sbody(i, _):
                rv = rowb_v[pl.ds(i * _L, _L)]
                dv = plsc.load_gather(dis_v, [rv])
                sb_v[pl.ds(i * _L, _L)] = dv * sb_v[pl.ds(i * _L, _L)]
                return 0
            lax.fori_loop(0, B // _L, sbody, 0)

            pltpu.sync_copy(colr_hbm.at[gid, pl.ds(bi * NCHB, NCHB)], colb_v)

            _gissue(0, msg0_v, gsem0)
            _gissue(1, msg1_v, gsem1)

            _stage(0, 0)
            _stage(1, 1)
            _stage(2, 2)

            def triple(t, _):
                k = 3 * t
                _stage(k, 0)
                _stage(k + 1, 1)
                _stage(k + 2, 2)
                return 0
            lax.fori_loop(1, NTRI - 1, triple, 0)

            kt = NCHB - 3
            _stage(kt, 0)
            for q, kk2 in ((1, kt + 1), (2, kt + 2)):
                mb, gs, ss = bufs[q]
                _gwait(kk2, mb, gs)
                _mult(mb, kk2)
                pltpu.sync_copy(mb, agg_sh.at[colb_v.at[kk2]], add=True)
            return 0
        lax.fori_loop(0, NBLK, block, 0)
        plsc.subcore_barrier()

        pltpu.async_copy(agg_sh.at[pl.ds(s * RPT, C)], msg0_v, gsem0)
        for k in range(NWS):
            cur, csem = (msg0_v, gsem0) if k % 2 == 0 else (msg1_v, gsem1)
            pltpu.make_async_copy(
                agg_sh.at[pl.ds(s * RPT + k * C, C)], cur, csem).wait()
            if k + 1 < NWS:
                nxt, nsem = (msg1_v, gsem1) if k % 2 == 0 else (msg0_v, gsem0)
                pltpu.async_copy(
                    agg_sh.at[pl.ds(s * RPT + (k + 1) * C, C)], nxt, nsem)
            pltpu.sync_copy(cur, part_hbm.at[c, pl.ds(s * RPT + k * C, C)])

    return agg_kernel


def _make_mm(N, D, BR):
    grid = N // BR

    def mm_body(x_ref, wpre_ref, bpre_ref, wlin_ref, blin_ref, wini_ref,
                wroot_ref, bias_ref, h1_ref, out0_ref, rootb_ref):
        xb = x_ref[...]
        h = lax.dot_general(xb, wpre_ref[...], (((1,), (1,)), ((), ())),
                            preferred_element_type=jnp.float32) + bpre_ref[...]
        h1 = lax.dot_general(h, wlin_ref[...], (((1,), (1,)), ((), ())),
                             preferred_element_type=jnp.float32) + blin_ref[...]
        h1 = jnp.where(h1 >= 0, h1, 0.01 * h1)
        h1_ref[...] = h1
        out0_ref[...] = jnp.dot(h1, wini_ref[...],
                                preferred_element_type=jnp.float32)
        rootb_ref[...] = jnp.dot(h1, wroot_ref[...],
                                 preferred_element_type=jnp.float32) + bias_ref[...]

    full = pl.BlockSpec((D, D), lambda i: (0, 0))
    bias = pl.BlockSpec((1, D), lambda i: (0, 0))
    rows = pl.BlockSpec((BR, D), lambda i: (i, 0))
    return pl.pallas_call(
        mm_body,
        grid=(grid,),
        in_specs=[rows, full, bias, full, bias, full, full, bias],
        out_specs=[rows, rows, rows],
        out_shape=[jax.ShapeDtypeStruct((N, D), jnp.float32)] * 3,
    )


def _make_epi(N, D, NPAD, BR):
    grid = N // BR
    rows = pl.BlockSpec((BR, D), lambda i: (i, 0))

    def epi_body(h1_ref, rootb_ref, p_ref, dis_ref, o_ref):
        ps = p_ref[0] + p_ref[1]
        a = jnp.maximum(ps * dis_ref[...] + rootb_ref[...], 0.0)
        o_ref[...] = jnp.tanh(h1_ref[...] + a)

    return pl.pallas_call(
        epi_body,
        grid=(grid,),
        in_specs=[
            rows, rows,
            pl.BlockSpec((2, BR, D), lambda i: (0, i, 0)),
            pl.BlockSpec((BR, 1), lambda i: (i, 0)),
        ],
        out_specs=rows,
        out_shape=jax.ShapeDtypeStruct((N, D), jnp.float32),
    )


def kernel(x, edge_index, edge_weight, W_pre, b_pre, W_lin, b_lin,
           arma_init_w, arma_root_w, arma_bias):
    N, D = x.shape
    E = edge_weight.shape[0]
    NPAD = ((N + _NW * _L - 1) // (_NW * _L)) * (_NW * _L)
    C = 64
    BLK = C * 24

    row = edge_index[0]
    col = edge_index[1]

    EPT = -(-E // (_NW * BLK)) * BLK
    EP = _NW * EPT
    if EP != E:
        pad = EP - E
        spread = (jnp.arange(pad, dtype=jnp.int32) * 8) % N
        row_p = jnp.concatenate([row, spread])
        col_p = jnp.concatenate([col, spread])
        ew_p = jnp.concatenate([edge_weight, jnp.zeros((pad,), jnp.float32)])
    else:
        row_p, col_p, ew_p = row, col, edge_weight
    colr = col_p.reshape(_NW, EPT // C, C)

    BR = 1000 if N % 1000 == 0 else N
    deg2 = _make_deg(E, NPAD)(col, edge_weight)
    h1, out0, rootb = _make_mm(N, D, BR)(
        x, W_pre, b_pre.reshape(1, D), W_lin, b_lin.reshape(1, D),
        arma_init_w, arma_root_w, arma_bias.reshape(1, D))
    part, dis = _make_agg(EP, N, D, NPAD, C)(row_p, colr, ew_p, deg2, out0)
    dis2d = dis.reshape(NPAD, 1)
    return _make_epi(N, D, NPAD, BR)(h1, rootb, part, dis2d)


# speedup vs baseline: 5.1285x; 1.1003x over previous
import functools

import jax
import jax.numpy as jnp
from jax import lax
from jax.experimental import pallas as pl
from jax.experimental.pallas import tpu as pltpu
from jax.experimental.pallas import tpu_sc as plsc

_NC, _NS, _L = 2, 16, 16
_NW = _NC * _NS


def _rsqrt_nr(d):
    i = plsc.bitcast(d, jnp.int32)
    i = jnp.int32(0x5F3759DF) - lax.shift_right_logical(i, 1)
    y = plsc.bitcast(i, jnp.float32)
    for _ in range(3):
        y = y * (1.5 - 0.5 * d * y * y)
    return jnp.where(d > 0.0, y, 0.0)


def _make_deg(E, NPAD):
    EPT = E // _NW
    NPS = NPAD // _NS
    mesh = plsc.VectorSubcoreMesh(
        core_axis_name="c", subcore_axis_name="s",
        num_cores=_NC, num_subcores=_NS)

    @functools.partial(
        pl.kernel,
        out_type=jax.ShapeDtypeStruct((_NC, NPAD), jnp.float32),
        mesh=mesh,
        scratch_types=[
            pltpu.VMEM((EPT,), jnp.int32),
            pltpu.VMEM((EPT,), jnp.float32),
            pltpu.VMEM((NPAD,), jnp.float32),
            pltpu.VMEM_SHARED((_NS, NPAD), jnp.float32),
            pltpu.VMEM((_NS, NPS), jnp.float32),
            pltpu.VMEM((NPS,), jnp.float32),
        ],
        compiler_params=pltpu.CompilerParams(needs_layout_passes=False),
    )
    def deg_kernel(col_hbm, ew_hbm, deg2_hbm, col_v, ew_v, part_v, stage_sh,
                   gath_v, out_v):
        c = lax.axis_index("c")
        s = lax.axis_index("s")
        gid = c * _NS + s
        zv = jnp.zeros((_L,), jnp.float32)

        def zbody(i, _):
            part_v[pl.ds(i * _L, _L)] = zv
            return 0
        lax.fori_loop(0, NPAD // _L, zbody, 0, unroll=4)

        pltpu.sync_copy(col_hbm.at[pl.ds(gid * EPT, EPT)], col_v)
        pltpu.sync_copy(ew_hbm.at[pl.ds(gid * EPT, EPT)], ew_v)

        def ebody(i, _):
            cv = col_v[pl.ds(i * _L, _L)]
            wv = ew_v[pl.ds(i * _L, _L)]
            plsc.addupdate_scatter(part_v, [cv], wv)
            return 0
        lax.fori_loop(0, EPT // _L, ebody, 0, unroll=4)

        pltpu.sync_copy(part_v, stage_sh.at[s])
        plsc.subcore_barrier()
        for t in range(_NS):
            pltpu.sync_copy(stage_sh.at[t, pl.ds(s * NPS, NPS)], gath_v.at[t])

        def cbody(j, _):
            acc = gath_v[0, pl.ds(j * _L, _L)]
            for t in range(1, _NS):
                acc = acc + gath_v[t, pl.ds(j * _L, _L)]
            out_v[pl.ds(j * _L, _L)] = acc
            return 0
        lax.fori_loop(0, NPS // _L, cbody, 0)
        pltpu.sync_copy(out_v, deg2_hbm.at[c, pl.ds(s * NPS, NPS)])

    return deg_kernel


def _make_agg(E, N, D, NPAD, C):
    EPT = E // _NW
    EPC = EPT // C
    NCHB = 24
    B = C * NCHB
    NBLK = EPT // B
    assert EPT % B == 0
    NTRI = NCHB // 3
    RPT = NPAD // _NS
    assert RPT % C == 0
    NWS = RPT // C
    DCH = 640 if NPAD % 640 == 0 else NPAD
    SB = max(B, 2 * DCH)
    NPS = NPAD // _NS
    mesh = plsc.VectorSubcoreMesh(
        core_axis_name="c", subcore_axis_name="s",
        num_cores=_NC, num_subcores=_NS)

    @functools.partial(
        pl.kernel,
        out_type=(jax.ShapeDtypeStruct((_NC, NPAD, D), jnp.float32),
                  jax.ShapeDtypeStruct((NPAD,), jnp.float32)),
        mesh=mesh,
        scratch_types=[
            pltpu.VMEM((NPAD,), jnp.float32),
            pltpu.VMEM((NCHB, C), jnp.int32),
            pltpu.VMEM((B,), jnp.int32),
            pltpu.VMEM((SB,), jnp.float32),
            pltpu.VMEM((C,), jnp.int32),
            pltpu.VMEM((C, D), jnp.float32),
            pltpu.VMEM((C, D), jnp.float32),
            pltpu.VMEM((C, D), jnp.float32),
            pltpu.VMEM_SHARED((NPAD, D), jnp.float32),
            pltpu.SemaphoreType.DMA,
            pltpu.SemaphoreType.DMA,
            pltpu.SemaphoreType.DMA,
            pltpu.SemaphoreType.DMA,
            pltpu.SemaphoreType.DMA,
            pltpu.SemaphoreType.DMA,
        ],
        compiler_params=pltpu.CompilerParams(needs_layout_passes=False),
    )
    def agg_kernel(row_hbm, colr_hbm, ew_hbm, deg2_hbm, out0_hbm,
                   part_hbm, dis_hbm,
                   dis_v, colb_v, rowb_v, sb_v, zidx_v, msg0_v, msg1_v, msg2_v,
                   agg_sh, gsem0, gsem1, gsem2, ssem0, ssem1, ssem2):
        c = lax.axis_index("c")
        s = lax.axis_index("s")
        gid = c * _NS + s
        bufs = ((msg0_v, gsem0, ssem0),
                (msg1_v, gsem1, ssem1),
                (msg2_v, gsem2, ssem2))

        def dchunk(b, _):
            pltpu.sync_copy(deg2_hbm.at[0, pl.ds(b * DCH, DCH)],
                            sb_v.at[pl.ds(0, DCH)])
            pltpu.sync_copy(deg2_hbm.at[1, pl.ds(b * DCH, DCH)],
                            sb_v.at[pl.ds(DCH, DCH)])

            def dbody(j, _):
                d = sb_v[pl.ds(j * _L, _L)] + sb_v[pl.ds(DCH + j * _L, _L)]
                dis_v[pl.ds(b * DCH + j * _L, _L)] = _rsqrt_nr(d)
                return 0
            lax.fori_loop(0, DCH // _L, dbody, 0)
            return 0
        lax.fori_loop(0, NPAD // DCH, dchunk, 0)

        @pl.when(c == 0)
        def _():
            pltpu.sync_copy(dis_v.at[pl.ds(s * NPS, NPS)],
                            dis_hbm.at[pl.ds(s * NPS, NPS)])

        zv = jnp.zeros((_L,), jnp.float32)
        zvi = jnp.zeros((_L,), jnp.int32)

        def zrow(i, _):
            for kk in range(D // _L):
                msg0_v[i, pl.ds(kk * _L, _L)] = zv
                msg1_v[i, pl.ds(kk * _L, _L)] = zv
                msg2_v[i, pl.ds(kk * _L, _L)] = zv
            return 0
        lax.fori_loop(0, C, zrow, 0)
        for i in range(C // _L):
            zidx_v[pl.ds(i * _L, _L)] = zvi

        def zagg(i, _):
            pltpu.sync_copy(msg0_v, agg_sh.at[pl.ds(s * RPT + i * C, C)])
            return 0
        lax.fori_loop(0, NWS, zagg, 0)

        plsc.subcore_barrier()

        base = gid * EPT

        def _mult(msg, k):
            def gbody(g, _):
                sv = sb_v[pl.ds(k * C + g * _L, _L)]
                for j in range(_L):
                    sc = sv[j]
                    for kk in range(D // _L):
                        msg[g * _L + j, pl.ds(kk * _L, _L)] = (
                            msg[g * _L + j, pl.ds(kk * _L, _L)] * sc)
                return 0
            lax.fori_loop(0, C // _L, gbody, 0)

        def _gissue(k, mb, gs):
            pltpu.async_copy(out0_hbm.at[rowb_v.at[pl.ds(k * C, C)]], mb, gs)

        def _gwait(k, mb, gs):
            pltpu.make_async_copy(
                out0_hbm.at[rowb_v.at[pl.ds(k * C, C)]], mb, gs).wait()

        def _swait(ss):
            pltpu.make_async_copy(msg0_v, agg_sh.at[zidx_v], ss).wait()

        def _swait(k, q):
            pltpu.make_async_copy(
                bufs[q][0], agg_sh.at[colb_v.at[k]], ssem0).wait()

        def _stage(k, q, first=False):
            mb, gs, ss = bufs[q]
            _gwait(k, mb, gs)
            _mult(mb, k)
            if not first:
                _swait(k - 1, (q + 2) % 3)
            pltpu.async_copy(mb, agg_sh.at[colb_v.at[k]], ssem0, add=True)
            pb, pgs, pss = bufs[(q + 2) % 3]
            _gissue(k + 2, pb, pgs)

        def block(bi, _):
            boff = base + bi * B
            pltpu.sync_copy(row_hbm.at[pl.ds(boff, B)], rowb_v)
            pltpu.sync_copy(ew_hbm.at[pl.ds(boff, B)], sb_v.at[pl.ds(0, B)])

            def sbody(i, _):
                rv = rowb_v[pl.ds(i * _L, _L)]
                dv = plsc.load_gather(dis_v, [rv])
                sb_v[pl.ds(i * _L, _L)] = dv * sb_v[pl.ds(i * _L, _L)]
                return 0
            lax.fori_loop(0, B // _L, sbody, 0)

            @pl.when(bi > 0)
            def _():
                _swait(NCHB - 1, 2)
            pltpu.sync_copy(colr_hbm.at[gid, pl.ds(bi * NCHB, NCHB)], colb_v)

            _gissue(0, msg0_v, gsem0)
            _gissue(1, msg1_v, gsem1)

            _stage(0, 0, first=True)
            _stage(1, 1)
            _stage(2, 2)

            def triple(t, _):
                k = 3 * t
                _stage(k, 0)
                _stage(k + 1, 1)
                _stage(k + 2, 2)
                return 0
            lax.fori_loop(1, NTRI - 1, triple, 0)

            kt = NCHB - 3
            _stage(kt, 0)
            for q, kk2 in ((1, kt + 1), (2, kt + 2)):
                mb, gs, ss = bufs[q]
                _gwait(kk2, mb, gs)
                _mult(mb, kk2)
                _swait(kk2 - 1, (q + 2) % 3)
                pltpu.async_copy(mb, agg_sh.at[colb_v.at[kk2]], ssem0,
                                 add=True)
            return 0
        lax.fori_loop(0, NBLK, block, 0)
        _swait(NCHB - 1, 2)
        plsc.subcore_barrier()

        pltpu.async_copy(agg_sh.at[pl.ds(s * RPT, C)], msg0_v, gsem0)
        for k in range(NWS):
            cur, csem = (msg0_v, gsem0) if k % 2 == 0 else (msg1_v, gsem1)
            pltpu.make_async_copy(
                agg_sh.at[pl.ds(s * RPT + k * C, C)], cur, csem).wait()
            if k + 1 < NWS:
                nxt, nsem = (msg1_v, gsem1) if k % 2 == 0 else (msg0_v, gsem0)
                pltpu.async_copy(
                    agg_sh.at[pl.ds(s * RPT + (k + 1) * C, C)], nxt, nsem)
            pltpu.sync_copy(cur, part_hbm.at[c, pl.ds(s * RPT + k * C, C)])

    return agg_kernel


def _make_mm(N, D, BR):
    grid = N // BR

    def mm_body(x_ref, wpre_ref, bpre_ref, wlin_ref, blin_ref, wini_ref,
                wroot_ref, bias_ref, h1_ref, out0_ref, rootb_ref):
        xb = x_ref[...]
        h = lax.dot_general(xb, wpre_ref[...], (((1,), (1,)), ((), ())),
                            preferred_element_type=jnp.float32) + bpre_ref[...]
        h1 = lax.dot_general(h, wlin_ref[...], (((1,), (1,)), ((), ())),
                             preferred_element_type=jnp.float32) + blin_ref[...]
        h1 = jnp.where(h1 >= 0, h1, 0.01 * h1)
        h1_ref[...] = h1
        out0_ref[...] = jnp.dot(h1, wini_ref[...],
                                preferred_element_type=jnp.float32)
        rootb_ref[...] = jnp.dot(h1, wroot_ref[...],
                                 preferred_element_type=jnp.float32) + bias_ref[...]

    full = pl.BlockSpec((D, D), lambda i: (0, 0))
    bias = pl.BlockSpec((1, D), lambda i: (0, 0))
    rows = pl.BlockSpec((BR, D), lambda i: (i, 0))
    return pl.pallas_call(
        mm_body,
        grid=(grid,),
        in_specs=[rows, full, bias, full, bias, full, full, bias],
        out_specs=[rows, rows, rows],
        out_shape=[jax.ShapeDtypeStruct((N, D), jnp.float32)] * 3,
    )


def _make_epi(N, D, NPAD, BR):
    grid = N // BR
    rows = pl.BlockSpec((BR, D), lambda i: (i, 0))

    def epi_body(h1_ref, rootb_ref, p_ref, dis_ref, o_ref):
        ps = p_ref[0] + p_ref[1]
        a = jnp.maximum(ps * dis_ref[...] + rootb_ref[...], 0.0)
        o_ref[...] = jnp.tanh(h1_ref[...] + a)

    return pl.pallas_call(
        epi_body,
        grid=(grid,),
        in_specs=[
            rows, rows,
            pl.BlockSpec((2, BR, D), lambda i: (0, i, 0)),
            pl.BlockSpec((BR, 1), lambda i: (i, 0)),
        ],
        out_specs=rows,
        out_shape=jax.ShapeDtypeStruct((N, D), jnp.float32),
    )


def kernel(x, edge_index, edge_weight, W_pre, b_pre, W_lin, b_lin,
           arma_init_w, arma_root_w, arma_bias):
    N, D = x.shape
    E = edge_weight.shape[0]
    NPAD = ((N + _NW * _L - 1) // (_NW * _L)) * (_NW * _L)
    C = 64
    BLK = C * 24

    row = edge_index[0]
    col = edge_index[1]

    EPT = -(-E // (_NW * BLK)) * BLK
    EP = _NW * EPT
    if EP != E:
        pad = EP - E
        spread = (jnp.arange(pad, dtype=jnp.int32) * 8) % N
        row_p = jnp.concatenate([row, spread])
        col_p = jnp.concatenate([col, spread])
        ew_p = jnp.concatenate([edge_weight, jnp.zeros((pad,), jnp.float32)])
    else:
        row_p, col_p, ew_p = row, col, edge_weight
    colr = col_p.reshape(_NW, EPT // C, C)

    BR = 1000 if N % 1000 == 0 else N
    deg2 = _make_deg(E, NPAD)(col, edge_weight)
    h1, out0, rootb = _make_mm(N, D, BR)(
        x, W_pre, b_pre.reshape(1, D), W_lin, b_lin.reshape(1, D),
        arma_init_w, arma_root_w, arma_bias.reshape(1, D))
    part, dis = _make_agg(EP, N, D, NPAD, C)(row_p, colr, ew_p, deg2, out0)
    dis2d = dis.reshape(NPAD, 1)
    return _make_epi(N, D, NPAD, BR)(h1, rootb, part, dis2d)


# speedup vs baseline: 5.2304x; 1.0199x over previous
import functools

import jax
import jax.numpy as jnp
from jax import lax
from jax.experimental import pallas as pl
from jax.experimental.pallas import tpu as pltpu
from jax.experimental.pallas import tpu_sc as plsc

_NC, _NS, _L = 2, 16, 16
_NW = _NC * _NS


def _rsqrt_nr(d):
    i = plsc.bitcast(d, jnp.int32)
    i = jnp.int32(0x5F3759DF) - lax.shift_right_logical(i, 1)
    y = plsc.bitcast(i, jnp.float32)
    for _ in range(3):
        y = y * (1.5 - 0.5 * d * y * y)
    return jnp.where(d > 0.0, y, 0.0)


def _make_deg(E, NPAD):
    EPT = E // _NW
    NPS = NPAD // _NS
    mesh = plsc.VectorSubcoreMesh(
        core_axis_name="c", subcore_axis_name="s",
        num_cores=_NC, num_subcores=_NS)

    @functools.partial(
        pl.kernel,
        out_type=jax.ShapeDtypeStruct((_NC, NPAD), jnp.float32),
        mesh=mesh,
        scratch_types=[
            pltpu.VMEM((EPT,), jnp.int32),
            pltpu.VMEM((EPT,), jnp.float32),
            pltpu.VMEM((NPAD,), jnp.float32),
            pltpu.VMEM_SHARED((_NS, NPAD), jnp.float32),
            pltpu.VMEM((_NS, NPS), jnp.float32),
            pltpu.VMEM((NPS,), jnp.float32),
        ],
        compiler_params=pltpu.CompilerParams(needs_layout_passes=False),
    )
    def deg_kernel(col_hbm, ew_hbm, deg2_hbm, col_v, ew_v, part_v, stage_sh,
                   gath_v, out_v):
        c = lax.axis_index("c")
        s = lax.axis_index("s")
        gid = c * _NS + s
        zv = jnp.zeros((_L,), jnp.float32)

        def zbody(i, _):
            part_v[pl.ds(i * _L, _L)] = zv
            return 0
        lax.fori_loop(0, NPAD // _L, zbody, 0, unroll=4)

        pltpu.sync_copy(col_hbm.at[pl.ds(gid * EPT, EPT)], col_v)
        pltpu.sync_copy(ew_hbm.at[pl.ds(gid * EPT, EPT)], ew_v)

        def ebody(i, _):
            cv = col_v[pl.ds(i * _L, _L)]
            wv = ew_v[pl.ds(i * _L, _L)]
            plsc.addupdate_scatter(part_v, [cv], wv)
            return 0
        lax.fori_loop(0, EPT // _L, ebody, 0, unroll=4)

        pltpu.sync_copy(part_v, stage_sh.at[s])
        plsc.subcore_barrier()
        for t in range(_NS):
            pltpu.sync_copy(stage_sh.at[t, pl.ds(s * NPS, NPS)], gath_v.at[t])

        def cbody(j, _):
            acc = gath_v[0, pl.ds(j * _L, _L)]
            for t in range(1, _NS):
                acc = acc + gath_v[t, pl.ds(j * _L, _L)]
            out_v[pl.ds(j * _L, _L)] = acc
            return 0
        lax.fori_loop(0, NPS // _L, cbody, 0)
        pltpu.sync_copy(out_v, deg2_hbm.at[c, pl.ds(s * NPS, NPS)])

    return deg_kernel


def _make_agg(E, N, D, NPAD, C):
    EPT = E // _NW
    EPC = EPT // C
    NCHB = 24
    B = C * NCHB
    NBLK = EPT // B
    assert EPT % B == 0
    NTRI = NCHB // 3
    RPT = NPAD // _NS
    assert RPT % C == 0
    NWS = RPT // C
    DCH = 640 if NPAD % 640 == 0 else NPAD
    SB = max(B, 2 * DCH)
    NPS = NPAD // _NS
    mesh = plsc.VectorSubcoreMesh(
        core_axis_name="c", subcore_axis_name="s",
        num_cores=_NC, num_subcores=_NS)

    @functools.partial(
        pl.kernel,
        out_type=(jax.ShapeDtypeStruct((_NC, NPAD, D), jnp.float32),
                  jax.ShapeDtypeStruct((NPAD,), jnp.float32)),
        mesh=mesh,
        scratch_types=[
            pltpu.VMEM((NPAD,), jnp.float32),
            pltpu.VMEM((NCHB, C), jnp.int32),
            pltpu.VMEM((B,), jnp.int32),
            pltpu.VMEM((SB,), jnp.float32),
            pltpu.VMEM((C,), jnp.int32),
            pltpu.VMEM((C, D), jnp.float32),
            pltpu.VMEM((C, D), jnp.float32),
            pltpu.VMEM((C, D), jnp.float32),
            pltpu.VMEM_SHARED((NPAD, D), jnp.float32),
            pltpu.SemaphoreType.DMA,
            pltpu.SemaphoreType.DMA,
            pltpu.SemaphoreType.DMA,
            pltpu.SemaphoreType.DMA,
            pltpu.SemaphoreType.DMA,
            pltpu.SemaphoreType.DMA,
        ],
        compiler_params=pltpu.CompilerParams(needs_layout_passes=False),
    )
    def agg_kernel(row_hbm, colr_hbm, ew_hbm, deg2_hbm, out0_hbm,
                   part_hbm, dis_hbm,
                   dis_v, colb_v, rowb_v, sb_v, zidx_v, msg0_v, msg1_v, msg2_v,
                   agg_sh, gsem0, gsem1, gsem2, ssem0, ssem1, ssem2):
        c = lax.axis_index("c")
        s = lax.axis_index("s")
        gid = c * _NS + s
        bufs = ((msg0_v, gsem0, ssem0),
                (msg1_v, gsem1, ssem1),
                (msg2_v, gsem2, ssem2))

        pltpu.sync_copy(deg2_hbm.at[0], dis_v)
        pltpu.async_copy(deg2_hbm.at[1, pl.ds(0, DCH)],
                         sb_v.at[pl.ds(0, DCH)], gsem0)

        def dchunk(b, _):
            pltpu.make_async_copy(deg2_hbm.at[1, pl.ds(b * DCH, DCH)],
                                  sb_v.at[pl.ds(0, DCH)], gsem0).wait()

            def dbody(j, _):
                d = dis_v[pl.ds(b * DCH + j * _L, _L)] + sb_v[pl.ds(j * _L, _L)]
                dis_v[pl.ds(b * DCH + j * _L, _L)] = _rsqrt_nr(d)
                return 0
            lax.fori_loop(0, DCH // _L, dbody, 0)

            @pl.when(b + 1 < NPAD // DCH)
            def _():
                pltpu.async_copy(deg2_hbm.at[1, pl.ds((b + 1) * DCH, DCH)],
                                 sb_v.at[pl.ds(0, DCH)], gsem0)
            return 0
        lax.fori_loop(0, NPAD // DCH, dchunk, 0)

        @pl.when(c == 0)
        def _():
            pltpu.sync_copy(dis_v.at[pl.ds(s * NPS, NPS)],
                            dis_hbm.at[pl.ds(s * NPS, NPS)])

        zv = jnp.zeros((_L,), jnp.float32)
        zvi = jnp.zeros((_L,), jnp.int32)

        def zrow(i, _):
            for kk in range(D // _L):
                msg0_v[i, pl.ds(kk * _L, _L)] = zv
                msg1_v[i, pl.ds(kk * _L, _L)] = zv
                msg2_v[i, pl.ds(kk * _L, _L)] = zv
            return 0
        lax.fori_loop(0, C, zrow, 0)
        for i in range(C // _L):
            zidx_v[pl.ds(i * _L, _L)] = zvi

        def zagg(i, _):
            pltpu.sync_copy(msg0_v, agg_sh.at[pl.ds(s * RPT + i * C, C)])
            return 0
        lax.fori_loop(0, NWS, zagg, 0)

        plsc.subcore_barrier()

        base = gid * EPT

        def _mult(msg, k):
            def gbody(g, _):
                sv = sb_v[pl.ds(k * C + g * _L, _L)]
                for j in range(_L):
                    sc = sv[j]
                    for kk in range(D // _L):
                        msg[g * _L + j, pl.ds(kk * _L, _L)] = (
                            msg[g * _L + j, pl.ds(kk * _L, _L)] * sc)
                return 0
            lax.fori_loop(0, C // _L, gbody, 0, unroll=2)

        def _gissue(k, mb, gs):
            pltpu.async_copy(out0_hbm.at[rowb_v.at[pl.ds(k * C, C)]], mb, gs)

        def _gwait(k, mb, gs):
            pltpu.make_async_copy(
                out0_hbm.at[rowb_v.at[pl.ds(k * C, C)]], mb, gs).wait()

        def _swait(ss):
            pltpu.make_async_copy(msg0_v, agg_sh.at[zidx_v], ss).wait()

        def _swait(k, q):
            pltpu.make_async_copy(
                bufs[q][0], agg_sh.at[colb_v.at[k]], ssem0).wait()

        def _stage(k, q, first=False):
            mb, gs, ss = bufs[q]
            _gwait(k, mb, gs)
            _mult(mb, k)
            if not first:
                _swait(k - 1, (q + 2) % 3)
            pltpu.async_copy(mb, agg_sh.at[colb_v.at[k]], ssem0, add=True)
            pb, pgs, pss = bufs[(q + 2) % 3]
            _gissue(k + 2, pb, pgs)

        def block(bi, _):
            boff = base + bi * B
            pltpu.sync_copy(row_hbm.at[pl.ds(boff, B)], rowb_v)
            pltpu.sync_copy(ew_hbm.at[pl.ds(boff, B)], sb_v.at[pl.ds(0, B)])

            def sbody(i, _):
                rv = rowb_v[pl.ds(i * _L, _L)]
                dv = plsc.load_gather(dis_v, [rv])
                sb_v[pl.ds(i * _L, _L)] = dv * sb_v[pl.ds(i * _L, _L)]
                return 0
            lax.fori_loop(0, B // _L, sbody, 0)

            @pl.when(bi > 0)
            def _():
                _swait(NCHB - 1, 2)
            pltpu.sync_copy(colr_hbm.at[gid, pl.ds(bi * NCHB, NCHB)], colb_v)

            _gissue(0, msg0_v, gsem0)
            _gissue(1, msg1_v, gsem1)

            _stage(0, 0, first=True)
            _stage(1, 1)
            _stage(2, 2)

            def triple(t, _):
                k = 3 * t
                _stage(k, 0)
                _stage(k + 1, 1)
                _stage(k + 2, 2)
                return 0
            lax.fori_loop(1, NTRI - 1, triple, 0)

            kt = NCHB - 3
            _stage(kt, 0)
            for q, kk2 in ((1, kt + 1), (2, kt + 2)):
                mb, gs, ss = bufs[q]
                _gwait(kk2, mb, gs)
                _mult(mb, kk2)
                _swait(kk2 - 1, (q + 2) % 3)
                pltpu.async_copy(mb, agg_sh.at[colb_v.at[kk2]], ssem0,
                                 add=True)
            return 0
        lax.fori_loop(0, NBLK, block, 0)
        _swait(NCHB - 1, 2)
        plsc.subcore_barrier()

        pltpu.async_copy(agg_sh.at[pl.ds(s * RPT, C)], msg0_v, gsem0)
        for k in range(NWS):
            cur, csem = (msg0_v, gsem0) if k % 2 == 0 else (msg1_v, gsem1)
            pltpu.make_async_copy(
                agg_sh.at[pl.ds(s * RPT + k * C, C)], cur, csem).wait()
            if k + 1 < NWS:
                nxt, nsem = (msg1_v, gsem1) if k % 2 == 0 else (msg0_v, gsem0)
                pltpu.async_copy(
                    agg_sh.at[pl.ds(s * RPT + (k + 1) * C, C)], nxt, nsem)
            pltpu.sync_copy(cur, part_hbm.at[c, pl.ds(s * RPT + k * C, C)])

    return agg_kernel


def _make_mm(N, D, BR):
    grid = N // BR

    def mm_body(x_ref, wpre_ref, bpre_ref, wlin_ref, blin_ref, wini_ref,
                wroot_ref, bias_ref, h1_ref, out0_ref, rootb_ref):
        xb = x_ref[...]
        h = lax.dot_general(xb, wpre_ref[...], (((1,), (1,)), ((), ())),
                            preferred_element_type=jnp.float32) + bpre_ref[...]
        h1 = lax.dot_general(h, wlin_ref[...], (((1,), (1,)), ((), ())),
                             preferred_element_type=jnp.float32) + blin_ref[...]
        h1 = jnp.where(h1 >= 0, h1, 0.01 * h1)
        h1_ref[...] = h1
        out0_ref[...] = jnp.dot(h1, wini_ref[...],
                                preferred_element_type=jnp.float32)
        rootb_ref[...] = jnp.dot(h1, wroot_ref[...],
                                 preferred_element_type=jnp.float32) + bias_ref[...]

    full = pl.BlockSpec((D, D), lambda i: (0, 0))
    bias = pl.BlockSpec((1, D), lambda i: (0, 0))
    rows = pl.BlockSpec((BR, D), lambda i: (i, 0))
    return pl.pallas_call(
        mm_body,
        grid=(grid,),
        in_specs=[rows, full, bias, full, bias, full, full, bias],
        out_specs=[rows, rows, rows],
        out_shape=[jax.ShapeDtypeStruct((N, D), jnp.float32)] * 3,
    )


def _make_epi(N, D, NPAD, BR):
    grid = N // BR
    rows = pl.BlockSpec((BR, D), lambda i: (i, 0))

    def epi_body(h1_ref, rootb_ref, p_ref, dis_ref, o_ref):
        ps = p_ref[0] + p_ref[1]
        a = jnp.maximum(ps * dis_ref[...] + rootb_ref[...], 0.0)
        o_ref[...] = jnp.tanh(h1_ref[...] + a)

    return pl.pallas_call(
        epi_body,
        grid=(grid,),
        in_specs=[
            rows, rows,
            pl.BlockSpec((2, BR, D), lambda i: (0, i, 0)),
            pl.BlockSpec((BR, 1), lambda i: (i, 0)),
        ],
        out_specs=rows,
        out_shape=jax.ShapeDtypeStruct((N, D), jnp.float32),
    )


def kernel(x, edge_index, edge_weight, W_pre, b_pre, W_lin, b_lin,
           arma_init_w, arma_root_w, arma_bias):
    N, D = x.shape
    E = edge_weight.shape[0]
    NPAD = ((N + _NW * _L - 1) // (_NW * _L)) * (_NW * _L)
    C = 64
    BLK = C * 24

    row = edge_index[0]
    col = edge_index[1]

    EPT = -(-E // (_NW * BLK)) * BLK
    EP = _NW * EPT
    if EP != E:
        pad = EP - E
        spread = (jnp.arange(pad, dtype=jnp.int32) * 8) % N
        row_p = jnp.concatenate([row, spread])
        col_p = jnp.concatenate([col, spread])
        ew_p = jnp.concatenate([edge_weight, jnp.zeros((pad,), jnp.float32)])
    else:
        row_p, col_p, ew_p = row, col, edge_weight
    colr = col_p.reshape(_NW, EPT // C, C)

    BR = 1000 if N % 1000 == 0 else N
    deg2 = _make_deg(E, NPAD)(col, edge_weight)
    h1, out0, rootb = _make_mm(N, D, BR)(
        x, W_pre, b_pre.reshape(1, D), W_lin, b_lin.reshape(1, D),
        arma_init_w, arma_root_w, arma_bias.reshape(1, D))
    part, dis = _make_agg(EP, N, D, NPAD, C)(row_p, colr, ew_p, deg2, out0)
    dis2d = dis.reshape(NPAD, 1)
    return _make_epi(N, D, NPAD, BR)(h1, rootb, part, dis2d)
